# baseline (speedup 1.0000x reference)
import functools

import jax
import jax.numpy as jnp
from jax import lax
from jax.experimental import pallas as pl
from jax.experimental.pallas import tpu as pltpu
from jax.experimental.pallas import tpu_sc as plsc

VOCAB = 100000
EMBED_DIM = 128
B_TOTAL = 4096 * 200

NC = 2
NS = 16
COLB = 16
NBLOCKS = EMBED_DIM // COLB
NPASS = NBLOCKS // NC

L_PER_TILE = B_TOTAL // NS
IDX_BLK = 2048
N_IDX_BLK = L_PER_TILE // IDX_BLK
CHUNK = 128
CHUNKS_PER_BLK = IDX_BLK // CHUNK
N_CHUNKS = L_PER_TILE // CHUNK
S = 4
P = 3

TROWS = 6400
TSTRIDE = 6240
TCHUNK = 640
TLOADS = TROWS // TCHUNK


def _make_gather():
    mesh = plsc.VectorSubcoreMesh(core_axis_name="c", subcore_axis_name="s")

    @functools.partial(
        pl.kernel,
        mesh=mesh,
        compiler_params=pltpu.CompilerParams(use_tc_tiling_on_sc=False),
        out_type=jax.ShapeDtypeStruct((B_TOTAL, NBLOCKS, COLB), jnp.float32),
        scratch_types=[
            pltpu.VMEM_SHARED((VOCAB, COLB), jnp.float32),
            pltpu.VMEM((TCHUNK, COLB), jnp.float32),
        ]
        + [pltpu.VMEM((IDX_BLK,), jnp.int32) for _ in range(2)]
        + [pltpu.SemaphoreType.DMA for _ in range(2)]
        + [pltpu.VMEM((CHUNK, COLB), jnp.float32) for _ in range(S)]
        + [pltpu.SemaphoreType.DMA for _ in range(2 * S)],
    )
    def k(w3_hbm, idx_hbm, out_hbm, sh, tbuf, ib0, ib1, is0, is1,
          *bufs_and_sems):
        ib = (ib0, ib1)
        isem = (is0, is1)
        gbuf = bufs_and_sems[:S]
        gsem = bufs_and_sems[S:2 * S]
        wsem = bufs_and_sems[2 * S:]

        sc = lax.axis_index("c")
        sid = lax.axis_index("s")
        base = sid * L_PER_TILE
        r0 = jnp.minimum(sid * TSTRIDE, VOCAB - TROWS)

        def load_table(cb):
            def body(c, carry):
                rs = r0 + c * TCHUNK
                pltpu.sync_copy(w3_hbm.at[pl.ds(rs, TCHUNK), cb, :], tbuf)
                pltpu.sync_copy(tbuf, sh.at[pl.ds(rs, TCHUNK)])
                return carry

            lax.fori_loop(0, TLOADS, body, 0)

        def idx_desc(blk, buf):
            return pltpu.make_async_copy(
                idx_hbm.at[pl.ds(base + blk * IDX_BLK, IDX_BLK)],
                ib[buf], isem[buf],
            )

        def g_desc(kpos, buf, s):
            return pltpu.make_async_copy(
                sh.at[ib[buf].at[pl.ds(kpos * CHUNK, CHUNK)]],
                gbuf[s], gsem[s],
            )

        def w_desc(q, cb, s):
            return pltpu.make_async_copy(
                gbuf[s],
                out_hbm.at[pl.ds(base + q * CHUNK, CHUNK), cb, :],
                wsem[s],
            )

        def run_pass(p):
            cb = 2 * p + sc
            load_table(cb)
            plsc.subcore_barrier()

            idx_desc(0, 0).start()
            idx_desc(0, 0).wait()
            for j in range(P):
                g_desc(j, 0, j % S).start()

            def do_step(q, kpos, buf, s, nbuf, first=False, last=False):
                g_desc(kpos, buf, s).wait()
                w_desc(q, cb, s).start()
                if not last:
                    if not first:
                        w_desc(q - 1, cb, (s + P) % S).wait()
                    g_desc((kpos + P) % CHUNKS_PER_BLK, nbuf,
                           (s + P) % S).start()

            def run_block(q0, blk_next, cur, first_blk=False,
                          last_blk=False):
                nxtb = (cur + 1) % 2
                if not last_blk:
                    idx_desc(blk_next, nxtb).start()
                for kpos in range(CHUNKS_PER_BLK):
                    q = q0 + kpos
                    tail = kpos >= CHUNKS_PER_BLK - P
                    if not last_blk and kpos == CHUNKS_PER_BLK - P:
                        idx_desc(blk_next, nxtb).wait()
                    do_step(
                        q, kpos, cur, kpos % S,
                        nbuf=(nxtb if tail else cur),
                        first=(first_blk and kpos == 0),
                        last=(last_blk and tail),
                    )

            run_block(0, 1, 0, first_blk=True)

            def pair_body(half, carry):
                blk = 2 * half + 1
                run_block(blk * CHUNKS_PER_BLK, blk + 1, 1)
                run_block((blk + 1) * CHUNKS_PER_BLK, blk + 2, 0)
                return carry

            lax.fori_loop(0, (N_IDX_BLK - 3) // 2, pair_body, 0)

            run_block((N_IDX_BLK - 2) * CHUNKS_PER_BLK, N_IDX_BLK - 1, 1)
            run_block((N_IDX_BLK - 1) * CHUNKS_PER_BLK, None, 0,
                      last_blk=True)

            for b in range(S):
                q = N_CHUNKS - S + b
                w_desc(q, cb, q % S).wait()
            plsc.subcore_barrier()

        for p in range(NPASS):
            run_pass(p)

    return k


_gather = _make_gather()


def kernel(doc, W):
    idx = doc.reshape(-1).astype(jnp.int32)
    w3 = W.reshape(VOCAB, NBLOCKS, COLB)
    out = _gather(w3, idx)
    return out.reshape(doc.shape[0], doc.shape[1], EMBED_DIM)


# speedup vs baseline: 2.1586x; 2.1586x over previous
import functools

import jax
import jax.numpy as jnp
from jax import lax
from jax.experimental import pallas as pl
from jax.experimental.pallas import tpu as pltpu
from jax.experimental.pallas import tpu_sc as plsc

VOCAB = 100000
EMBED_DIM = 128
B_TOTAL = 4096 * 200

NC = 2
NS = 16
NW = NC * NS
B_PER_W = B_TOTAL // NW
CHUNK = 128
N_CHUNKS = B_PER_W // CHUNK
S = 5
P = 3


def _make_gather():
    mesh = plsc.VectorSubcoreMesh(core_axis_name="c", subcore_axis_name="s")

    @functools.partial(
        pl.kernel,
        mesh=mesh,
        out_type=jax.ShapeDtypeStruct((B_TOTAL, EMBED_DIM), jnp.float32),
        scratch_types=[
            pltpu.VMEM((B_PER_W,), jnp.int32),
        ]
        + [pltpu.VMEM((CHUNK, EMBED_DIM), jnp.float32) for _ in range(S)]
        + [pltpu.SemaphoreType.DMA for _ in range(2 * S)],
    )
    def k(table_hbm, idx_hbm, out_hbm, idx_v, *bufs_and_sems):
        rows = bufs_and_sems[:S]
        gsem = bufs_and_sems[S:2 * S]
        wsem = bufs_and_sems[2 * S:]
        wid = lax.axis_index("s") * NC + lax.axis_index("c")
        base = wid * B_PER_W

        pltpu.sync_copy(idx_hbm.at[pl.ds(base, B_PER_W)], idx_v)

        def issue_g(j, s):
            pltpu.async_copy(
                table_hbm.at[idx_v.at[pl.ds(j * CHUNK, CHUNK)]], rows[s], gsem[s]
            )

        def wait_g(i, s):
            pltpu.make_async_copy(
                table_hbm.at[idx_v.at[pl.ds(i * CHUNK, CHUNK)]], rows[s], gsem[s]
            ).wait()

        def issue_w(i, s):
            pltpu.async_copy(
                rows[s], out_hbm.at[pl.ds(base + i * CHUNK, CHUNK)], wsem[s]
            )

        def wait_w(m, s):
            pltpu.make_async_copy(
                rows[s], out_hbm.at[pl.ds(base + m * CHUNK, CHUNK)], wsem[s]
            ).wait()

        def step(i, s, do_wait_w, do_issue_g):
            wait_g(i, s)
            issue_w(i, s)
            j = i + P
            sj = (s + P) % S
            if do_wait_w:
                wait_w(j - S, sj)
            if do_issue_g:
                issue_g(j, sj)

        for j in range(P):
            issue_g(j, j % S)
        for i in range(S):
            step(i, i % S, i + P >= S, True)

        def body(g, carry):
            i0 = g * S
            for b in range(S):
                step(i0 + b, b, True, True)
            return carry

        lax.fori_loop(1, N_CHUNKS // S - 1, body, 0)

        for b in range(S):
            i = N_CHUNKS - S + b
            live = i + P < N_CHUNKS
            step(i, b, live, live)
        for b in range(S):
            wait_w(N_CHUNKS - S + b, b)

    return k


_gather = _make_gather()


def kernel(doc, W):
    idx = doc.reshape(-1).astype(jnp.int32)
    out = _gather(W, idx)
    return out.reshape(doc.shape[0], doc.shape[1], EMBED_DIM)
